# baseline (speedup 1.0000x reference)
import functools
import jax
import jax.numpy as jnp
from jax import lax
from jax.experimental import pallas as pl
from jax.experimental.pallas import tpu as pltpu
from jax.experimental.pallas import tpu_sc as plsc

DIM_W = 100000
HID = 32
BATCH = 16384
NCAT = 26
B_TOT = BATCH * NCAT
L_TOT = B_TOT // 4
NC = 2
NS = 16
NW = NC * NS
L_PER_W = L_TOT // NW
CHUNK_L = 256
N_CHUNKS = L_PER_W // CHUNK_L
NBUF = 3

_mesh = plsc.VectorSubcoreMesh(core_axis_name="c", subcore_axis_name="s")


@functools.partial(
    pl.kernel,
    mesh=_mesh,
    out_type=jax.ShapeDtypeStruct((L_TOT, 128), jnp.float32),
    scratch_types=[
        pltpu.VMEM((4, L_PER_W), jnp.int32),
        pltpu.VMEM((NBUF, 4, CHUNK_L, HID), jnp.float32),
        pltpu.SemaphoreType.DMA,
        pltpu.SemaphoreType.DMA,
    ],
    compiler_params=pltpu.CompilerParams(use_tc_tiling_on_sc=False),
)
def _sc_gather(idxt_hbm, w_hbm, out_hbm, idx_v, rows_v, gsem, ssem):
    wid = lax.axis_index("s") * NC + lax.axis_index("c")
    base = wid * L_PER_W
    for k in range(4):
        pltpu.sync_copy(
            idxt_hbm.at[pl.ds(k * L_TOT + base, L_PER_W)], idx_v.at[k]
        )

    def fire_gathers(i):
        return [
            pltpu.async_copy(
                w_hbm.at[idx_v.at[k, pl.ds(i * CHUNK_L, CHUNK_L)]],
                rows_v.at[i % NBUF, k],
                gsem,
            )
            for k in range(4)
        ]

    def fire_stores(i):
        return [
            pltpu.async_copy(
                rows_v.at[i % NBUF, k],
                out_hbm.at[
                    pl.ds(base + i * CHUNK_L, CHUNK_L), pl.ds(k * HID, HID)
                ],
                ssem,
            )
            for k in range(4)
        ]

    gathers = [fire_gathers(i) for i in range(min(NBUF, N_CHUNKS))]
    stores = []
    for i in range(N_CHUNKS):
        for g in gathers[i]:
            g.wait()
        stores.append(fire_stores(i))
        j = i + NBUF
        if j < N_CHUNKS:
            for s in stores[i]:
                s.wait()
            gathers.append(fire_gathers(j))
    for i in range(max(0, N_CHUNKS - NBUF), N_CHUNKS):
        for s in stores[i]:
            s.wait()


def kernel(x, W):
    idx = x[:, 0, :].reshape(L_TOT, 4)
    idxt = idx.T.reshape(4 * L_TOT)
    out = _sc_gather(idxt, W)
    return out.reshape(BATCH, NCAT, HID)


# speedup vs baseline: 1.3633x; 1.3633x over previous
import functools
import jax
import jax.numpy as jnp
from jax import lax
from jax.experimental import pallas as pl
from jax.experimental.pallas import tpu as pltpu
from jax.experimental.pallas import tpu_sc as plsc

DIM_W = 100000
HID = 32
BATCH = 16384
NCAT = 26
B_TOT = BATCH * NCAT
NC = 2
NS = 16
NW = NC * NS
B_PER_W = B_TOT // NW
CHUNK = 1024
N_CHUNKS = B_PER_W // CHUNK
NBUF = 3

_mesh = plsc.VectorSubcoreMesh(core_axis_name="c", subcore_axis_name="s")


@functools.partial(
    pl.kernel,
    mesh=_mesh,
    out_type=jax.ShapeDtypeStruct((B_TOT, HID), jnp.float32),
    scratch_types=[
        pltpu.VMEM((B_PER_W,), jnp.int32),
        pltpu.VMEM((NBUF, CHUNK, HID), jnp.float32),
        pltpu.SemaphoreType.DMA,
        pltpu.SemaphoreType.DMA,
    ],
    compiler_params=pltpu.CompilerParams(use_tc_tiling_on_sc=False),
)
def _sc_gather(idx_hbm, w_hbm, out_hbm, idx_v, rows_v, gsem, ssem):
    wid = lax.axis_index("s") * NC + lax.axis_index("c")
    base = wid * B_PER_W
    pltpu.sync_copy(idx_hbm.at[pl.ds(base, B_PER_W)], idx_v)

    def fire_gather(i):
        return pltpu.async_copy(
            w_hbm.at[idx_v.at[pl.ds(i * CHUNK, CHUNK)]],
            rows_v.at[i % NBUF],
            gsem,
        )

    gathers = [fire_gather(i) for i in range(min(NBUF, N_CHUNKS))]
    stores = []
    for i in range(N_CHUNKS):
        gathers[i].wait()
        stores.append(
            pltpu.async_copy(
                rows_v.at[i % NBUF],
                out_hbm.at[pl.ds(base + i * CHUNK, CHUNK)],
                ssem,
            )
        )
        j = i + NBUF
        if j < N_CHUNKS:
            stores[i].wait()
            gathers.append(fire_gather(j))
    for i in range(max(0, N_CHUNKS - NBUF), N_CHUNKS):
        stores[i].wait()


def kernel(x, W):
    idx_cb = x[:, 0, :].T.reshape(B_TOT)
    out = _sc_gather(idx_cb, W)
    return out.reshape(NCAT, BATCH, HID).transpose(1, 0, 2)
